# baseline (speedup 1.0000x reference)
import functools

import jax
import jax.numpy as jnp
from jax import lax
from jax.experimental import pallas as pl
from jax.experimental.pallas import tpu as pltpu
from jax.experimental.pallas import tpu_sc as plsc

_SEQ = 199
_D = 64
_BB = 256
_HROWS = 208


def _sc_gather_rows(table, idx):
    n = idx.shape[0]
    d = table.shape[1]
    info = plsc.get_sparse_core_info()
    nw = info.num_cores * info.num_subcores
    ch = 128
    chunks_total = -(-n // ch)
    chunks_w = -(-chunks_total // nw)
    pad = chunks_w * nw * ch
    idx_p = jnp.pad(idx.astype(jnp.int32), (0, pad - n))

    mesh = plsc.VectorSubcoreMesh(core_axis_name="c", subcore_axis_name="s")

    @functools.partial(
        pl.kernel,
        mesh=mesh,
        out_type=jax.ShapeDtypeStruct((pad, d), jnp.float32),
        scratch_types=[
            pltpu.VMEM((ch,), jnp.int32),
            pltpu.VMEM((ch, d), jnp.float32),
            pltpu.SemaphoreType.DMA,
        ],
        compiler_params=pltpu.CompilerParams(use_tc_tiling_on_sc=False),
    )
    def gather_k(table_hbm, idx_hbm, out_hbm, idx_v, rows_v, sem):
        wid = lax.axis_index("s") * info.num_cores + lax.axis_index("c")
        base = wid * (chunks_w * ch)

        def body(c, carry):
            off = base + c * ch
            pltpu.sync_copy(idx_hbm.at[pl.ds(off, ch)], idx_v)
            pltpu.async_copy(table_hbm.at[idx_v], rows_v, sem).wait()
            pltpu.sync_copy(rows_v, out_hbm.at[pl.ds(off, ch)])
            return carry

        lax.fori_loop(0, chunks_w, body, 0)

    return gather_k(table, idx_p)


def _rekt_step_kernel(g_ref, skl_ref, ans_ref, init0_ref, ls0_ref, wbig_ref,
                      bbig_ref, w2_ref, b2_ref, asf_ref, sfw2_ref, sfb_ref,
                      aaf_ref, afw2_ref, afb_ref, asw2_ref, ssw2_ref, aet_ref,
                      tet_ref, out_ref, h_ref, tgc_ref):
    f32 = jnp.float32
    tgc_ref[...] = jnp.dot(sfw2_ref[...], tet_ref[...], preferred_element_type=f32)
    caf = jnp.dot(afw2_ref[...], tet_ref[:, 1:2], preferred_element_type=f32) + afb_ref[...]
    aas = jnp.dot(asw2_ref[...], aet_ref[...], preferred_element_type=f32)
    ass = jnp.dot(ssw2_ref[...], aet_ref[...], preferred_element_type=f32)
    h_ref[...] = jnp.zeros((_HROWS, _D, _BB), f32)
    h_ref[0] = jnp.broadcast_to(init0_ref[...], (_D, _BB))
    ast0 = jnp.broadcast_to(ls0_ref[...], (_D, _BB))
    wbig = wbig_ref[...]
    bbig = bbig_ref[...]
    asf = asf_ref[...]
    aaf = aaf_ref[...]
    w2c = w2_ref[...]
    b2 = b2_ref[...]
    sfb = sfb_ref[...]

    def lt_gsf(t):
        skl_all = skl_ref[...]
        skl_t = skl_ref[t]
        it3 = lax.broadcasted_iota(jnp.int32, (_SEQ, 1, _BB), 0)
        match = (skl_all == skl_t[None]) & (it3 < t)
        lt = jnp.max(jnp.where(match, it3, 0), axis=0)
        oh = (lax.broadcasted_iota(jnp.int32, (200, _BB), 0)
              == jnp.broadcast_to(t - lt, (200, _BB))).astype(f32)
        gsf = jnp.dot(tgc_ref[...], oh, preferred_element_type=f32)
        return lt, gsf

    def scan_ls(t, lt):
        def chunk(c, acc):
            off = pl.multiple_of(c * 16, 8)
            for k in range(16):
                mk = (c * 16 + k) == lt
                acc = jnp.where(mk, h_ref[off + k], acc)
            return acc

        return lax.fori_loop(0, (t - 1) // 16 + 1, chunk,
                             jnp.zeros((_D, _BB), f32))

    lt0, gsf0 = lt_gsf(0)
    ls_pre0 = jnp.broadcast_to(init0_ref[...], (_D, _BB))

    def step(t, carry):
        ast, lt, gsf, ls = carry

        sf = jax.nn.sigmoid(jnp.dot(asf, ls, preferred_element_type=f32) + gsf + sfb)
        ls2 = ls * sf
        af = jax.nn.sigmoid(jnp.dot(aaf, ast, preferred_element_type=f32) + caf)
        la = ast * af
        pe = g_ref[t]
        x = jnp.concatenate([la, ls2, pe], axis=0)
        y = jnp.dot(wbig, x, preferred_element_type=f32) + bbig
        hh = jax.nn.relu(y[0:_D])
        ans_t = ans_ref[t].astype(f32)
        tas = aas[:, 0:1] + ans_t * (aas[:, 1:2] - aas[:, 0:1])
        tss = ass[:, 0:1] + ans_t * (ass[:, 1:2] - ass[:, 0:1])
        ast_new = la + jnp.tanh(y[_D:2 * _D] + tas)
        iss = ls2 + jnp.tanh(y[2 * _D:3 * _D] + tss)
        h_ref[t] = iss
        p = jax.nn.sigmoid(jnp.sum(hh * w2c, axis=0, keepdims=True) + b2)
        out_ref[t] = p
        tn = jnp.minimum(t + 1, _SEQ - 1)
        ltn, gsfn = lt_gsf(tn)
        lsn = scan_ls(tn, ltn)
        return ast_new, ltn, gsfn, lsn

    lax.fori_loop(0, _SEQ, step, (ast0, lt0, gsf0, ls_pre0))


def kernel(last_problem, last_skill, last_ans, next_problem, next_skill,
           next_ans, skill_embed, ans_embed, time_embed, ls_state,
           skill_state_init, out_W1, out_b1, out_W2, out_b2, sf_W, sf_b,
           af_W, af_b, ss_W, ss_b, as_W, as_b):
    f32 = jnp.float32
    skl = next_skill.astype(jnp.int32)
    ans = next_ans.astype(jnp.int32)
    b, seq = skl.shape
    d = skill_embed.shape[1]

    idx = skl.T.reshape(-1)
    rows = _sc_gather_rows(skill_embed.astype(f32), idx)[: seq * b]
    gT = rows.reshape(seq, b, d).transpose(0, 2, 1)

    sklT = skl.T.reshape(seq, 1, b)
    ansT = ans.T.reshape(seq, 1, b)

    wbig = jnp.concatenate([
        out_W1,
        jnp.concatenate([as_W[:, :d], jnp.zeros((d, d), f32), as_W[:, d:]], axis=1),
        jnp.concatenate([jnp.zeros((d, d), f32), ss_W[:, :d], ss_W[:, d:]], axis=1),
    ], axis=0)
    bbig = jnp.concatenate([out_b1, as_b, ss_b]).reshape(3 * d, 1)

    args = (gT, sklT, ansT,
            skill_state_init[0].reshape(d, 1),
            ls_state.reshape(d, 1),
            wbig, bbig,
            out_W2.reshape(d, 1),
            out_b2.reshape(1, 1),
            sf_W[:, :d], sf_W[:, d:], sf_b.reshape(d, 1),
            af_W[:, :d], af_W[:, d:], af_b.reshape(d, 1),
            as_W[:, d:], ss_W[:, d:],
            ans_embed.T,
            time_embed.T)

    def full(s):
        return pl.BlockSpec(s, lambda *_: tuple(0 for _ in s))

    in_specs = [
        pl.BlockSpec((seq, d, _BB), lambda i: (0, 0, i)),
        pl.BlockSpec((seq, 1, _BB), lambda i: (0, 0, i)),
        pl.BlockSpec((seq, 1, _BB), lambda i: (0, 0, i)),
        full((d, 1)), full((d, 1)), full((3 * d, 3 * d)), full((3 * d, 1)),
        full((d, 1)), full((1, 1)),
        full((d, d)), full((d, d)), full((d, 1)),
        full((d, d)), full((d, d)), full((d, 1)),
        full((d, d)), full((d, d)),
        full((d, 2)), full((d, 200)),
    ]
    out = pl.pallas_call(
        _rekt_step_kernel,
        grid=(b // _BB,),
        in_specs=in_specs,
        out_specs=pl.BlockSpec((seq, 1, _BB), lambda i: (0, 0, i)),
        out_shape=jax.ShapeDtypeStruct((seq, 1, b), f32),
        scratch_shapes=[pltpu.VMEM((_HROWS, _D, _BB), f32),
                        pltpu.VMEM((_D, 200), f32)],
        compiler_params=pltpu.CompilerParams(
            dimension_semantics=("arbitrary",)),
    )(*args)
    return out.reshape(seq, b).T


# speedup vs baseline: 1.1070x; 1.1070x over previous
import functools

import jax
import jax.numpy as jnp
from jax import lax
from jax.experimental import pallas as pl
from jax.experimental.pallas import tpu as pltpu
from jax.experimental.pallas import tpu_sc as plsc

_SEQ = 199
_D = 64
_BB = 256
_HROWS = 208


def _sc_gather_rows(table, idx):
    n = idx.shape[0]
    d = table.shape[1]
    info = plsc.get_sparse_core_info()
    nw = info.num_cores * info.num_subcores
    ch = 128
    chunks_total = -(-n // ch)
    chunks_w = -(-chunks_total // nw)
    pad = chunks_w * nw * ch
    idx_p = jnp.pad(idx.astype(jnp.int32), (0, pad - n))

    mesh = plsc.VectorSubcoreMesh(core_axis_name="c", subcore_axis_name="s")

    @functools.partial(
        pl.kernel,
        mesh=mesh,
        out_type=jax.ShapeDtypeStruct((pad, d), jnp.float32),
        scratch_types=[
            pltpu.VMEM((ch,), jnp.int32),
            pltpu.VMEM((ch, d), jnp.float32),
            pltpu.SemaphoreType.DMA,
        ],
        compiler_params=pltpu.CompilerParams(use_tc_tiling_on_sc=False),
    )
    def gather_k(table_hbm, idx_hbm, out_hbm, idx_v, rows_v, sem):
        wid = lax.axis_index("s") * info.num_cores + lax.axis_index("c")
        base = wid * (chunks_w * ch)

        def body(c, carry):
            off = base + c * ch
            pltpu.sync_copy(idx_hbm.at[pl.ds(off, ch)], idx_v)
            pltpu.async_copy(table_hbm.at[idx_v], rows_v, sem).wait()
            pltpu.sync_copy(rows_v, out_hbm.at[pl.ds(off, ch)])
            return carry

        lax.fori_loop(0, chunks_w, body, 0)

    return gather_k(table, idx_p)


def _rekt_step_kernel(g_ref, skl_ref, ans_ref, init0_ref, ls0_ref, wbig_ref,
                      bbig_ref, w2_ref, b2_ref, asf_ref, sfw2_ref, sfb_ref,
                      aaf_ref, afw2_ref, afb_ref, asw2_ref, ssw2_ref, aet_ref,
                      tet_ref, out_ref, h_ref, tgc_ref):
    f32 = jnp.float32
    tgc_ref[...] = jnp.dot(sfw2_ref[...], tet_ref[...], preferred_element_type=f32)
    caf = jnp.dot(afw2_ref[...], tet_ref[:, 1:2], preferred_element_type=f32) + afb_ref[...]
    aas = jnp.dot(asw2_ref[...], aet_ref[...], preferred_element_type=f32)
    ass = jnp.dot(ssw2_ref[...], aet_ref[...], preferred_element_type=f32)
    bf16 = jnp.bfloat16
    h_ref[...] = jnp.zeros((_HROWS, _D, _BB), bf16)
    h_ref[0] = jnp.broadcast_to(init0_ref[...], (_D, _BB)).astype(bf16)
    ast0 = jnp.broadcast_to(ls0_ref[...], (_D, _BB))
    wbig = wbig_ref[...]
    bbig = bbig_ref[...]
    asf = asf_ref[...]
    aaf = aaf_ref[...]
    w2c = w2_ref[...]
    b2 = b2_ref[...]
    sfb = sfb_ref[...]

    def lt_gsf(t):
        skl_all = skl_ref[...]
        skl_t = skl_ref[t]
        it3 = lax.broadcasted_iota(jnp.int32, (_SEQ, 1, _BB), 0)
        match = (skl_all == skl_t[None]) & (it3 < t)
        lt = jnp.max(jnp.where(match, it3, 0), axis=0)
        oh = (lax.broadcasted_iota(jnp.int32, (200, _BB), 0)
              == jnp.broadcast_to(t - lt, (200, _BB))).astype(f32)
        gsf = jnp.dot(tgc_ref[...], oh, preferred_element_type=f32)
        return lt, gsf

    def scan_ls(t, lt):
        def chunk(c, acc):
            off = pl.multiple_of(c * 16, 8)
            for k in range(16):
                mk = (c * 16 + k) == lt
                acc = jnp.where(mk, h_ref[off + k], acc)
            return acc

        return lax.fori_loop(0, (t - 1) // 16 + 1, chunk,
                             jnp.zeros((_D, _BB), jnp.bfloat16))

    lt0, gsf0 = lt_gsf(0)
    ls_pre0 = jnp.broadcast_to(init0_ref[...], (_D, _BB)).astype(jnp.bfloat16)

    def step(t, carry):
        ast, lt, gsf, lsb = carry
        ls = lsb.astype(f32)

        sf = jax.nn.sigmoid(jnp.dot(asf, ls, preferred_element_type=f32) + gsf + sfb)
        ls2 = ls * sf
        af = jax.nn.sigmoid(jnp.dot(aaf, ast, preferred_element_type=f32) + caf)
        la = ast * af
        pe = g_ref[t]
        x = jnp.concatenate([la, ls2, pe], axis=0)
        y = jnp.dot(wbig, x, preferred_element_type=f32) + bbig
        hh = jax.nn.relu(y[0:_D])
        ans_t = ans_ref[t].astype(f32)
        tas = aas[:, 0:1] + ans_t * (aas[:, 1:2] - aas[:, 0:1])
        tss = ass[:, 0:1] + ans_t * (ass[:, 1:2] - ass[:, 0:1])
        ast_new = la + jnp.tanh(y[_D:2 * _D] + tas)
        iss = ls2 + jnp.tanh(y[2 * _D:3 * _D] + tss)
        h_ref[t] = iss.astype(bf16)
        p = jax.nn.sigmoid(jnp.sum(hh * w2c, axis=0, keepdims=True) + b2)
        out_ref[t] = p
        tn = jnp.minimum(t + 1, _SEQ - 1)
        ltn, gsfn = lt_gsf(tn)
        lsn = scan_ls(tn, ltn)
        return ast_new, ltn, gsfn, lsn

    lax.fori_loop(0, _SEQ, step, (ast0, lt0, gsf0, ls_pre0))


def kernel(last_problem, last_skill, last_ans, next_problem, next_skill,
           next_ans, skill_embed, ans_embed, time_embed, ls_state,
           skill_state_init, out_W1, out_b1, out_W2, out_b2, sf_W, sf_b,
           af_W, af_b, ss_W, ss_b, as_W, as_b):
    f32 = jnp.float32
    skl = next_skill.astype(jnp.int32)
    ans = next_ans.astype(jnp.int32)
    b, seq = skl.shape
    d = skill_embed.shape[1]

    idx = skl.T.reshape(-1)
    rows = _sc_gather_rows(skill_embed.astype(f32), idx)[: seq * b]
    gT = rows.reshape(seq, b, d).transpose(0, 2, 1)

    sklT = skl.T.reshape(seq, 1, b)
    ansT = ans.T.reshape(seq, 1, b)

    wbig = jnp.concatenate([
        out_W1,
        jnp.concatenate([as_W[:, :d], jnp.zeros((d, d), f32), as_W[:, d:]], axis=1),
        jnp.concatenate([jnp.zeros((d, d), f32), ss_W[:, :d], ss_W[:, d:]], axis=1),
    ], axis=0)
    bbig = jnp.concatenate([out_b1, as_b, ss_b]).reshape(3 * d, 1)

    args = (gT, sklT, ansT,
            skill_state_init[0].reshape(d, 1),
            ls_state.reshape(d, 1),
            wbig, bbig,
            out_W2.reshape(d, 1),
            out_b2.reshape(1, 1),
            sf_W[:, :d], sf_W[:, d:], sf_b.reshape(d, 1),
            af_W[:, :d], af_W[:, d:], af_b.reshape(d, 1),
            as_W[:, d:], ss_W[:, d:],
            ans_embed.T,
            time_embed.T)

    def full(s):
        return pl.BlockSpec(s, lambda *_: tuple(0 for _ in s))

    in_specs = [
        pl.BlockSpec((seq, d, _BB), lambda i: (0, 0, i)),
        pl.BlockSpec((seq, 1, _BB), lambda i: (0, 0, i)),
        pl.BlockSpec((seq, 1, _BB), lambda i: (0, 0, i)),
        full((d, 1)), full((d, 1)), full((3 * d, 3 * d)), full((3 * d, 1)),
        full((d, 1)), full((1, 1)),
        full((d, d)), full((d, d)), full((d, 1)),
        full((d, d)), full((d, d)), full((d, 1)),
        full((d, d)), full((d, d)),
        full((d, 2)), full((d, 200)),
    ]
    out = pl.pallas_call(
        _rekt_step_kernel,
        grid=(b // _BB,),
        in_specs=in_specs,
        out_specs=pl.BlockSpec((seq, 1, _BB), lambda i: (0, 0, i)),
        out_shape=jax.ShapeDtypeStruct((seq, 1, b), f32),
        scratch_shapes=[pltpu.VMEM((_HROWS, _D, _BB), jnp.bfloat16),
                        pltpu.VMEM((_D, 200), f32)],
        compiler_params=pltpu.CompilerParams(
            dimension_semantics=("arbitrary",)),
    )(*args)
    return out.reshape(seq, b).T


# speedup vs baseline: 1.1199x; 1.0116x over previous
import functools

import jax
import jax.numpy as jnp
from jax import lax
from jax.experimental import pallas as pl
from jax.experimental.pallas import tpu as pltpu
from jax.experimental.pallas import tpu_sc as plsc

_SEQ = 199
_D = 64
_BB = 256
_HROWS = 208


def _sc_gather_rows(table, idx):
    n = idx.shape[0]
    d = table.shape[1]
    info = plsc.get_sparse_core_info()
    nw = info.num_cores * info.num_subcores
    ch = 128
    chunks_total = -(-n // ch)
    chunks_w = -(-chunks_total // nw)
    chunks_w += chunks_w % 2
    pad = chunks_w * nw * ch
    idx_p = jnp.pad(idx.astype(jnp.int32), (0, pad - n))

    mesh = plsc.VectorSubcoreMesh(core_axis_name="c", subcore_axis_name="s")

    @functools.partial(
        pl.kernel,
        mesh=mesh,
        out_type=jax.ShapeDtypeStruct((pad, d), jnp.float32),
        scratch_types=[
            pltpu.VMEM((chunks_w * ch,), jnp.int32),
            pltpu.VMEM((ch, d), jnp.float32),
            pltpu.VMEM((ch, d), jnp.float32),
            pltpu.SemaphoreType.DMA,
            pltpu.SemaphoreType.DMA,
        ],
        compiler_params=pltpu.CompilerParams(use_tc_tiling_on_sc=False),
    )
    def gather_k(table_hbm, idx_hbm, out_hbm, idx_v, rows_a, rows_b, sem_a,
                 sem_b):
        wid = lax.axis_index("s") * info.num_cores + lax.axis_index("c")
        base = wid * (chunks_w * ch)
        pltpu.sync_copy(idx_hbm.at[pl.ds(base, chunks_w * ch)], idx_v)
        pltpu.async_copy(table_hbm.at[idx_v.at[pl.ds(0, ch)]], rows_a, sem_a)

        def pair(i, carry):
            c0 = i * 2
            pltpu.async_copy(
                table_hbm.at[idx_v.at[pl.ds((c0 + 1) * ch, ch)]], rows_b,
                sem_b)
            pltpu.make_async_copy(out_hbm.at[pl.ds(base, ch)], rows_a,
                                  sem_a).wait()
            pltpu.sync_copy(rows_a, out_hbm.at[pl.ds(base + c0 * ch, ch)])

            @pl.when(c0 + 2 < chunks_w)
            def _():
                pltpu.async_copy(
                    table_hbm.at[idx_v.at[pl.ds((c0 + 2) * ch, ch)]], rows_a,
                    sem_a)

            pltpu.make_async_copy(out_hbm.at[pl.ds(base, ch)], rows_b,
                                  sem_b).wait()
            pltpu.sync_copy(rows_b,
                            out_hbm.at[pl.ds(base + (c0 + 1) * ch, ch)])
            return carry

        lax.fori_loop(0, chunks_w // 2, pair, 0)

    return gather_k(table, idx_p)


def _rekt_step_kernel(g_ref, skl_ref, ans_ref, init0_ref, ls0_ref, wab_ref,
                      wc_ref, bbig_ref, w2_ref, b2_ref, asf_ref, sfw2_ref, sfb_ref,
                      aaf_ref, afw2_ref, afb_ref, asw2_ref, ssw2_ref, aet_ref,
                      tet_ref, out_ref, h_ref, tgc_ref):
    f32 = jnp.float32
    tgc_ref[...] = jnp.dot(sfw2_ref[...], tet_ref[...], preferred_element_type=f32)
    caf = jnp.dot(afw2_ref[...], tet_ref[:, 1:2], preferred_element_type=f32) + afb_ref[...]
    aas = jnp.dot(asw2_ref[...], aet_ref[...], preferred_element_type=f32)
    ass = jnp.dot(ssw2_ref[...], aet_ref[...], preferred_element_type=f32)
    bf16 = jnp.bfloat16
    h_ref[...] = jnp.zeros((_HROWS, _D, _BB), bf16)
    h_ref[0] = jnp.broadcast_to(init0_ref[...], (_D, _BB)).astype(bf16)
    ast0 = jnp.broadcast_to(ls0_ref[...], (_D, _BB))
    wab = wab_ref[...]
    wc = wc_ref[...]
    bbig = bbig_ref[...]
    asf = asf_ref[...]
    aaf = aaf_ref[...]
    w2c = w2_ref[...]
    b2 = b2_ref[...]
    sfb = sfb_ref[...]

    def lt_gsf(t):
        skl_all = skl_ref[...]
        skl_t = skl_ref[t]
        it3 = lax.broadcasted_iota(jnp.int32, (_SEQ, 1, _BB), 0)
        match = (skl_all == skl_t[None]) & (it3 < t)
        lt = jnp.max(jnp.where(match, it3, 0), axis=0)
        oh = (lax.broadcasted_iota(jnp.int32, (200, _BB), 0)
              == jnp.broadcast_to(t - lt, (200, _BB))).astype(f32)
        gsf = jnp.dot(tgc_ref[...], oh, preferred_element_type=f32)
        return lt, gsf

    def scan_ls(t, lt):
        def chunk(c, acc):
            off = pl.multiple_of(c * 16, 8)
            for k in range(16):
                mk = (c * 16 + k) == lt
                acc = jnp.where(mk, h_ref[off + k], acc)
            return acc

        return lax.fori_loop(0, (t - 1) // 16 + 1, chunk,
                             jnp.zeros((_D, _BB), jnp.bfloat16))

    lt0, gsf0 = lt_gsf(0)
    ls_pre0 = jnp.broadcast_to(init0_ref[...], (_D, _BB)).astype(jnp.bfloat16)

    def step(t, carry):
        ast, lt, gsf, lsb = carry
        ls = lsb.astype(f32)

        sf = jax.nn.sigmoid(jnp.dot(asf, ls, preferred_element_type=f32) + gsf + sfb)
        ls2 = ls * sf
        af = jax.nn.sigmoid(jnp.dot(aaf, ast, preferred_element_type=f32) + caf)
        la = ast * af
        pe = g_ref[t]
        x = jnp.concatenate([la, ls2], axis=0)
        y = (jnp.dot(wab, x, preferred_element_type=f32)
             + jnp.dot(wc, pe, preferred_element_type=f32)
             + bbig)
        hh = jax.nn.relu(y[0:_D])
        ans_t = ans_ref[t].astype(f32)
        tas = aas[:, 0:1] + ans_t * (aas[:, 1:2] - aas[:, 0:1])
        tss = ass[:, 0:1] + ans_t * (ass[:, 1:2] - ass[:, 0:1])
        ast_new = la + jnp.tanh(y[_D:2 * _D] + tas)
        iss = ls2 + jnp.tanh(y[2 * _D:3 * _D] + tss)
        h_ref[t] = iss.astype(bf16)
        p = jax.nn.sigmoid(jnp.sum(hh * w2c, axis=0, keepdims=True) + b2)
        out_ref[t] = p
        tn = jnp.minimum(t + 1, _SEQ - 1)
        ltn, gsfn = lt_gsf(tn)
        lsn = scan_ls(tn, ltn)
        return ast_new, ltn, gsfn, lsn

    lax.fori_loop(0, _SEQ, step, (ast0, lt0, gsf0, ls_pre0))


def kernel(last_problem, last_skill, last_ans, next_problem, next_skill,
           next_ans, skill_embed, ans_embed, time_embed, ls_state,
           skill_state_init, out_W1, out_b1, out_W2, out_b2, sf_W, sf_b,
           af_W, af_b, ss_W, ss_b, as_W, as_b):
    f32 = jnp.float32
    skl = next_skill.astype(jnp.int32)
    ans = next_ans.astype(jnp.int32)
    b, seq = skl.shape
    d = skill_embed.shape[1]

    idx = skl.T.reshape(-1)
    rows = _sc_gather_rows(skill_embed.astype(f32), idx)[: seq * b]
    gT = rows.reshape(seq, b, d).transpose(0, 2, 1)

    sklT = skl.T.reshape(seq, 1, b)
    ansT = ans.T.reshape(seq, 1, b)

    wbig = jnp.concatenate([
        out_W1,
        jnp.concatenate([as_W[:, :d], jnp.zeros((d, d), f32), as_W[:, d:]], axis=1),
        jnp.concatenate([jnp.zeros((d, d), f32), ss_W[:, :d], ss_W[:, d:]], axis=1),
    ], axis=0)
    bbig = jnp.concatenate([out_b1, as_b, ss_b]).reshape(3 * d, 1)

    args = (gT, sklT, ansT,
            skill_state_init[0].reshape(d, 1),
            ls_state.reshape(d, 1),
            wbig[:, :2 * d], wbig[:, 2 * d:], bbig,
            out_W2.reshape(d, 1),
            out_b2.reshape(1, 1),
            sf_W[:, :d], sf_W[:, d:], sf_b.reshape(d, 1),
            af_W[:, :d], af_W[:, d:], af_b.reshape(d, 1),
            as_W[:, d:], ss_W[:, d:],
            ans_embed.T,
            time_embed.T)

    def full(s):
        return pl.BlockSpec(s, lambda *_: tuple(0 for _ in s))

    in_specs = [
        pl.BlockSpec((seq, d, _BB), lambda i: (0, 0, i)),
        pl.BlockSpec((seq, 1, _BB), lambda i: (0, 0, i)),
        pl.BlockSpec((seq, 1, _BB), lambda i: (0, 0, i)),
        full((d, 1)), full((d, 1)), full((3 * d, 2 * d)), full((3 * d, d)),
        full((3 * d, 1)),
        full((d, 1)), full((1, 1)),
        full((d, d)), full((d, d)), full((d, 1)),
        full((d, d)), full((d, d)), full((d, 1)),
        full((d, d)), full((d, d)),
        full((d, 2)), full((d, 200)),
    ]
    out = pl.pallas_call(
        _rekt_step_kernel,
        grid=(b // _BB,),
        in_specs=in_specs,
        out_specs=pl.BlockSpec((seq, 1, _BB), lambda i: (0, 0, i)),
        out_shape=jax.ShapeDtypeStruct((seq, 1, b), f32),
        scratch_shapes=[pltpu.VMEM((_HROWS, _D, _BB), jnp.bfloat16),
                        pltpu.VMEM((_D, 200), f32)],
        compiler_params=pltpu.CompilerParams(
            dimension_semantics=("arbitrary",)),
    )(*args)
    return out.reshape(seq, b).T
